# Initial kernel scaffold; baseline (speedup 1.0000x reference)
import jax
import jax.numpy as jnp
from jax.experimental import pallas as pl


def kernel(embeddings, edge_index, W1, b1, W2, b2):
    raise NotImplementedError("write your pallas kernel here")


# speedup vs baseline: 21.1069x; 21.1069x over previous
import functools

import jax
import jax.numpy as jnp
from jax import lax
from jax.experimental import pallas as pl
from jax.experimental.pallas import tpu as pltpu
from jax.experimental.pallas import tpu_sc as plsc

N = 10000
D = 128
E = 320000

NC = 2
NS = 16
NW = NC * NS
NP = 10240
CH = 128
EPT = 10240
EP = NW * EPT
NCH = EPT // CH
RPT = NP // NS

_MESH = plsc.VectorSubcoreMesh(core_axis_name="c", subcore_axis_name="s")


@functools.partial(
    pl.kernel,
    out_type=jax.ShapeDtypeStruct((NC, NP), jnp.float32),
    mesh=_MESH,
    scratch_types=[
        pltpu.VMEM((NCH, CH), jnp.int32),
        pltpu.VMEM((CH,), jnp.float32),
        pltpu.VMEM((RPT,), jnp.float32),
        pltpu.VMEM_SHARED((NP,), jnp.float32),
    ],
)
def _deg_call(dst_hbm, out_hbm, dst_v, ones_v, zeros_v, acc_sh):
    c = lax.axis_index("c")
    s = lax.axis_index("s")
    wid = c * NS + s

    row0 = pl.multiple_of(wid * NCH, NCH)
    pltpu.sync_copy(dst_hbm.at[pl.ds(row0, NCH)], dst_v)

    def fill(i, _):
        o = pl.multiple_of(i * 16, 16)
        ones_v[pl.ds(o, 16)] = jnp.ones((16,), jnp.float32)
        return 0
    lax.fori_loop(0, CH // 16, fill, 0)

    def zfill(i, _):
        o = pl.multiple_of(i * 16, 16)
        zeros_v[pl.ds(o, 16)] = jnp.zeros((16,), jnp.float32)
        return 0
    lax.fori_loop(0, RPT // 16, zfill, 0)

    a0 = pl.multiple_of(s * RPT, RPT)
    pltpu.sync_copy(zeros_v, acc_sh.at[pl.ds(a0, RPT)])
    plsc.subcore_barrier()

    def body(j, _):
        pltpu.sync_copy(ones_v, acc_sh.at[dst_v.at[j]], add=True)
        return 0
    lax.fori_loop(0, NCH, body, 0)

    plsc.subcore_barrier()
    pltpu.sync_copy(acc_sh.at[pl.ds(a0, RPT)], out_hbm.at[c, pl.ds(a0, RPT)])


@functools.partial(
    pl.kernel,
    out_type=jax.ShapeDtypeStruct((NC, NP, D), jnp.float32),
    mesh=_MESH,
    scratch_types=[
        pltpu.VMEM((EPT,), jnp.int32),
        pltpu.VMEM((NCH, CH), jnp.int32),
        pltpu.VMEM((CH, D), jnp.float32),
        pltpu.VMEM_SHARED((NP, D), jnp.float32),
        pltpu.SemaphoreType.DMA,
    ],
)
def _agg_call(hs_hbm, src_hbm, dst_hbm, out_hbm, src_v, dst_v, rows_v, acc_sh, sem):
    c = lax.axis_index("c")
    s = lax.axis_index("s")
    wid = c * NS + s

    e0 = pl.multiple_of(wid * EPT, EPT)
    pltpu.sync_copy(src_hbm.at[pl.ds(e0, EPT)], src_v)
    row0 = pl.multiple_of(wid * NCH, NCH)
    pltpu.sync_copy(dst_hbm.at[pl.ds(row0, NCH)], dst_v)

    def zrow(i, _):
        r = i // 8
        o = pl.multiple_of((i - r * 8) * 16, 16)
        rows_v[r, pl.ds(o, 16)] = jnp.zeros((16,), jnp.float32)
        return 0
    lax.fori_loop(0, CH * (D // 16), zrow, 0)

    def zcopy(i, _):
        o = pl.multiple_of(s * RPT + i * CH, CH)
        pltpu.sync_copy(rows_v, acc_sh.at[pl.ds(o, CH)])
        return 0
    lax.fori_loop(0, RPT // CH, zcopy, 0)
    plsc.subcore_barrier()

    def body(j, _):
        o = pl.multiple_of(j * CH, CH)
        pltpu.async_copy(hs_hbm.at[src_v.at[pl.ds(o, CH)]], rows_v, sem).wait()
        pltpu.sync_copy(rows_v, acc_sh.at[dst_v.at[j]], add=True)
        return 0
    lax.fori_loop(0, NCH, body, 0)

    plsc.subcore_barrier()
    a0 = pl.multiple_of(s * RPT, RPT)
    pltpu.sync_copy(acc_sh.at[pl.ds(a0, RPT)], out_hbm.at[c, pl.ds(a0, RPT)])


BN = 512
GRID = NP // BN


def _mm1_body(degp_ref, x_ref, w_ref, hs_ref, dis_ref):
    degp = degp_ref[...]
    ones = jnp.ones((NC, 1), jnp.float32)
    deg = lax.dot_general(degp, ones, (((0,), (0,)), ((), ())),
                          preferred_element_type=jnp.float32) + 1.0
    dis = lax.rsqrt(deg)
    h = jnp.dot(x_ref[...], w_ref[...], preferred_element_type=jnp.float32)
    hs_ref[...] = h * dis
    dis_ref[...] = dis


def _mm2_body(a0_ref, a1_ref, hs_ref, dis_ref, b_ref, w_ref, hs2_ref):
    dis = dis_ref[...]
    out1 = dis * (a0_ref[...] + a1_ref[...] + hs_ref[...]) + b_ref[...]
    h2 = jnp.dot(out1, w_ref[...], preferred_element_type=jnp.float32)
    hs2_ref[...] = h2 * dis


def _fin_body(a0_ref, a1_ref, hs_ref, dis_ref, b_ref, o_ref):
    o_ref[...] = (dis_ref[...] * (a0_ref[...] + a1_ref[...] + hs_ref[...])
                  + b_ref[...])


_row_spec = pl.BlockSpec((BN, D), lambda i: (i, 0))
_dis_spec = pl.BlockSpec((BN, 1), lambda i: (i, 0))
_w_spec = pl.BlockSpec((D, D), lambda i: (0, 0))
_b_spec = pl.BlockSpec((1, D), lambda i: (0, 0))

_mm1 = pl.pallas_call(
    _mm1_body,
    grid=(GRID,),
    in_specs=[pl.BlockSpec((NC, BN), lambda i: (0, i)), _row_spec, _w_spec],
    out_specs=[_row_spec, _dis_spec],
    out_shape=[jax.ShapeDtypeStruct((NP, D), jnp.float32),
               jax.ShapeDtypeStruct((NP, 1), jnp.float32)],
)

_mm2 = pl.pallas_call(
    _mm2_body,
    grid=(GRID,),
    in_specs=[_row_spec, _row_spec, _row_spec, _dis_spec, _b_spec, _w_spec],
    out_specs=_row_spec,
    out_shape=jax.ShapeDtypeStruct((NP, D), jnp.float32),
)

_fin = pl.pallas_call(
    _fin_body,
    grid=(GRID,),
    in_specs=[_row_spec, _row_spec, _row_spec, _dis_spec, _b_spec],
    out_specs=_row_spec,
    out_shape=jax.ShapeDtypeStruct((NP, D), jnp.float32),
)


def kernel(embeddings, edge_index, W1, b1, W2, b2):
    pad_idx = N + (jnp.arange(EP - E, dtype=jnp.int32) % (NP - N))
    src = jnp.concatenate([edge_index[0], pad_idx])
    dst2d = jnp.concatenate([edge_index[1], pad_idx]).reshape(EP // CH, CH)
    xpad = jnp.pad(embeddings, ((0, NP - N), (0, 0)))

    degp = _deg_call(dst2d)
    hs1, dis = _mm1(degp, xpad, W1)
    acc1 = _agg_call(hs1, src, dst2d)
    hs2 = _mm2(acc1[0], acc1[1], hs1, dis, b1.reshape(1, D), W2)
    acc2 = _agg_call(hs2, src, dst2d)
    out = _fin(acc2[0], acc2[1], hs2, dis, b2.reshape(1, D))
    return out[:N]


# speedup vs baseline: 26.1704x; 1.2399x over previous
import functools

import jax
import jax.numpy as jnp
from jax import lax
from jax.experimental import pallas as pl
from jax.experimental.pallas import tpu as pltpu
from jax.experimental.pallas import tpu_sc as plsc

N = 10000
D = 128
E = 320000

NC = 2
NS = 16
NW = NC * NS
NP = 10240
CH = 64
EPT = 10240
EP = NW * EPT
NCH = EPT // CH
RPT = NP // NS

_MESH = plsc.VectorSubcoreMesh(core_axis_name="c", subcore_axis_name="s")


@functools.partial(
    pl.kernel,
    out_type=jax.ShapeDtypeStruct((NC, NP), jnp.float32),
    mesh=_MESH,
    scratch_types=[
        pltpu.VMEM((NCH, CH), jnp.int32),
        pltpu.VMEM((CH,), jnp.float32),
        pltpu.VMEM((RPT,), jnp.float32),
        pltpu.VMEM_SHARED((NP,), jnp.float32),
    ],
)
def _deg_call(dst_hbm, out_hbm, dst_v, ones_v, zeros_v, acc_sh):
    c = lax.axis_index("c")
    s = lax.axis_index("s")
    wid = c * NS + s

    row0 = pl.multiple_of(wid * NCH, NCH)
    pltpu.sync_copy(dst_hbm.at[pl.ds(row0, NCH)], dst_v)

    def fill(i, _):
        o = pl.multiple_of(i * 16, 16)
        ones_v[pl.ds(o, 16)] = jnp.ones((16,), jnp.float32)
        return 0
    lax.fori_loop(0, CH // 16, fill, 0)

    def zfill(i, _):
        o = pl.multiple_of(i * 16, 16)
        zeros_v[pl.ds(o, 16)] = jnp.zeros((16,), jnp.float32)
        return 0
    lax.fori_loop(0, RPT // 16, zfill, 0)

    a0 = pl.multiple_of(s * RPT, RPT)
    pltpu.sync_copy(zeros_v, acc_sh.at[pl.ds(a0, RPT)])
    plsc.subcore_barrier()

    def body(j, _):
        pltpu.sync_copy(ones_v, acc_sh.at[dst_v.at[j]], add=True)
        return 0
    lax.fori_loop(0, NCH, body, 0)

    plsc.subcore_barrier()
    pltpu.sync_copy(acc_sh.at[pl.ds(a0, RPT)], out_hbm.at[c, pl.ds(a0, RPT)])


@functools.partial(
    pl.kernel,
    out_type=jax.ShapeDtypeStruct((NC, NP, D), jnp.float32),
    mesh=_MESH,
    scratch_types=[
        pltpu.VMEM((EPT,), jnp.int32),
        pltpu.VMEM((NCH, CH), jnp.int32),
        pltpu.VMEM((CH, D), jnp.float32),
        pltpu.VMEM((CH, D), jnp.float32),
        pltpu.VMEM_SHARED((NP, D), jnp.float32),
        pltpu.SemaphoreType.DMA,
        pltpu.SemaphoreType.DMA,
    ],
)
def _agg_call(hs_hbm, src_hbm, dst_hbm, out_hbm,
              src_v, dst_v, rows0_v, rows1_v, acc_sh, sem0, sem1):
    c = lax.axis_index("c")
    s = lax.axis_index("s")
    wid = c * NS + s

    e0 = pl.multiple_of(wid * EPT, EPT)
    pltpu.sync_copy(src_hbm.at[pl.ds(e0, EPT)], src_v)
    row0 = pl.multiple_of(wid * NCH, NCH)
    pltpu.sync_copy(dst_hbm.at[pl.ds(row0, NCH)], dst_v)

    def zrow(i, _):
        r = i // 8
        o = pl.multiple_of((i - r * 8) * 16, 16)
        rows0_v[r, pl.ds(o, 16)] = jnp.zeros((16,), jnp.float32)
        return 0
    lax.fori_loop(0, CH * (D // 16), zrow, 0)

    def zcopy(i, _):
        o = pl.multiple_of(s * RPT + i * CH, CH)
        pltpu.sync_copy(rows0_v, acc_sh.at[pl.ds(o, CH)])
        return 0
    lax.fori_loop(0, RPT // CH, zcopy, 0)
    plsc.subcore_barrier()

    bufs = ((rows0_v, sem0), (rows1_v, sem1))

    def gather(j, rows, sem):
        o = pl.multiple_of(j * CH, CH)
        return pltpu.async_copy(hs_hbm.at[src_v.at[pl.ds(o, CH)]], rows, sem)

    gather(0, *bufs[0])
    gather(1, *bufs[1])

    def body(i, _):
        for b, (rows, sem) in enumerate(bufs):
            j = i * 2 + b
            pltpu.make_async_copy(hs_hbm.at[src_v.at[pl.ds(0, CH)]], rows, sem).wait()
            pltpu.sync_copy(rows, acc_sh.at[dst_v.at[j]], add=True)

            @pl.when(j + 2 < NCH)
            def _():
                gather(j + 2, rows, sem)
        return 0
    lax.fori_loop(0, NCH // 2, body, 0)

    plsc.subcore_barrier()
    a0 = pl.multiple_of(s * RPT, RPT)
    pltpu.sync_copy(acc_sh.at[pl.ds(a0, RPT)], out_hbm.at[c, pl.ds(a0, RPT)])


BN = 512
GRID = NP // BN


def _mm1_body(degp_ref, x_ref, w_ref, hs_ref, dis_ref):
    degp = degp_ref[...]
    ones = jnp.ones((NC, 1), jnp.float32)
    deg = lax.dot_general(degp, ones, (((0,), (0,)), ((), ())),
                          preferred_element_type=jnp.float32) + 1.0
    dis = lax.rsqrt(deg)
    h = jnp.dot(x_ref[...], w_ref[...], preferred_element_type=jnp.float32)
    hs_ref[...] = h * dis
    dis_ref[...] = dis


def _mm2_body(a0_ref, a1_ref, hs_ref, dis_ref, b_ref, w_ref, hs2_ref):
    dis = dis_ref[...]
    out1 = dis * (a0_ref[...] + a1_ref[...] + hs_ref[...]) + b_ref[...]
    h2 = jnp.dot(out1, w_ref[...], preferred_element_type=jnp.float32)
    hs2_ref[...] = h2 * dis


def _fin_body(a0_ref, a1_ref, hs_ref, dis_ref, b_ref, o_ref):
    o_ref[...] = (dis_ref[...] * (a0_ref[...] + a1_ref[...] + hs_ref[...])
                  + b_ref[...])


_row_spec = pl.BlockSpec((BN, D), lambda i: (i, 0))
_dis_spec = pl.BlockSpec((BN, 1), lambda i: (i, 0))
_w_spec = pl.BlockSpec((D, D), lambda i: (0, 0))
_b_spec = pl.BlockSpec((1, D), lambda i: (0, 0))

_mm1 = pl.pallas_call(
    _mm1_body,
    grid=(GRID,),
    in_specs=[pl.BlockSpec((NC, BN), lambda i: (0, i)), _row_spec, _w_spec],
    out_specs=[_row_spec, _dis_spec],
    out_shape=[jax.ShapeDtypeStruct((NP, D), jnp.float32),
               jax.ShapeDtypeStruct((NP, 1), jnp.float32)],
)

_mm2 = pl.pallas_call(
    _mm2_body,
    grid=(GRID,),
    in_specs=[_row_spec, _row_spec, _row_spec, _dis_spec, _b_spec, _w_spec],
    out_specs=_row_spec,
    out_shape=jax.ShapeDtypeStruct((NP, D), jnp.float32),
)

_BF = 400
_fin_row = pl.BlockSpec((_BF, D), lambda i: (i, 0))
_fin = pl.pallas_call(
    _fin_body,
    grid=(N // _BF,),
    in_specs=[_fin_row, _fin_row, _fin_row,
              pl.BlockSpec((_BF, 1), lambda i: (i, 0)), _b_spec],
    out_specs=_fin_row,
    out_shape=jax.ShapeDtypeStruct((N, D), jnp.float32),
)


def kernel(embeddings, edge_index, W1, b1, W2, b2):
    pad_idx = N + (jnp.arange(EP - E, dtype=jnp.int32) % (NP - N))
    src = jnp.concatenate([edge_index[0], pad_idx])
    dst2d = jnp.concatenate([edge_index[1], pad_idx]).reshape(EP // CH, CH)
    xpad = jnp.pad(embeddings, ((0, NP - N), (0, 0)))

    degp = _deg_call(dst2d)
    hs1, dis = _mm1(degp, xpad, W1)
    acc1 = _agg_call(hs1, src, dst2d)
    hs2 = _mm2(acc1[0], acc1[1], hs1, dis, b1.reshape(1, D), W2)
    acc2 = _agg_call(hs2, src, dst2d)
    return _fin(acc2[0], acc2[1], hs2, dis, b2.reshape(1, D))


# speedup vs baseline: 27.8115x; 1.0627x over previous
import functools

import jax
import jax.numpy as jnp
from jax import lax
from jax.experimental import pallas as pl
from jax.experimental.pallas import tpu as pltpu
from jax.experimental.pallas import tpu_sc as plsc

N = 10000
D = 128
E = 320000

NC = 2
NS = 16
NW = NC * NS
NP = 10240
CH = 80
EPT = 10240
EP = NW * EPT
NCH = EPT // CH
RPT = NP // NS

_MESH = plsc.VectorSubcoreMesh(core_axis_name="c", subcore_axis_name="s")


@functools.partial(
    pl.kernel,
    out_type=jax.ShapeDtypeStruct((NC, NP), jnp.float32),
    mesh=_MESH,
    scratch_types=[
        pltpu.VMEM((NCH, CH), jnp.int32),
        pltpu.VMEM((CH,), jnp.float32),
        pltpu.VMEM((RPT,), jnp.float32),
        pltpu.VMEM_SHARED((NP,), jnp.float32),
    ],
)
def _deg_call(dst_hbm, out_hbm, dst_v, ones_v, zeros_v, acc_sh):
    c = lax.axis_index("c")
    s = lax.axis_index("s")
    wid = c * NS + s

    row0 = pl.multiple_of(wid * NCH, NCH)
    pltpu.sync_copy(dst_hbm.at[pl.ds(row0, NCH)], dst_v)

    def fill(i, _):
        o = pl.multiple_of(i * 16, 16)
        ones_v[pl.ds(o, 16)] = jnp.ones((16,), jnp.float32)
        return 0
    lax.fori_loop(0, CH // 16, fill, 0)

    def zfill(i, _):
        o = pl.multiple_of(i * 16, 16)
        zeros_v[pl.ds(o, 16)] = jnp.zeros((16,), jnp.float32)
        return 0
    lax.fori_loop(0, RPT // 16, zfill, 0)

    a0 = pl.multiple_of(s * RPT, RPT)
    pltpu.sync_copy(zeros_v, acc_sh.at[pl.ds(a0, RPT)])
    plsc.subcore_barrier()

    def body(j, _):
        pltpu.sync_copy(ones_v, acc_sh.at[dst_v.at[j]], add=True)
        return 0
    lax.fori_loop(0, NCH, body, 0)

    plsc.subcore_barrier()
    pltpu.sync_copy(acc_sh.at[pl.ds(a0, RPT)], out_hbm.at[c, pl.ds(a0, RPT)])


@functools.partial(
    pl.kernel,
    out_type=jax.ShapeDtypeStruct((NC, NP, D), jnp.float32),
    mesh=_MESH,
    scratch_types=[
        pltpu.VMEM((EPT,), jnp.int32),
        pltpu.VMEM((NCH, CH), jnp.int32),
        pltpu.VMEM((CH, D), jnp.float32),
        pltpu.VMEM((CH, D), jnp.float32),
        pltpu.VMEM_SHARED((NP, D), jnp.float32),
        pltpu.SemaphoreType.DMA,
        pltpu.SemaphoreType.DMA,
    ],
)
def _agg_call(hs_hbm, src_hbm, dst_hbm, out_hbm,
              src_v, dst_v, rows0_v, rows1_v, acc_sh, sem0, sem1):
    c = lax.axis_index("c")
    s = lax.axis_index("s")
    wid = c * NS + s

    e0 = pl.multiple_of(wid * EPT, EPT)
    pltpu.sync_copy(src_hbm.at[pl.ds(e0, EPT)], src_v)
    row0 = pl.multiple_of(wid * NCH, NCH)
    pltpu.sync_copy(dst_hbm.at[pl.ds(row0, NCH)], dst_v)

    def zrow(i, _):
        r = i // 8
        o = pl.multiple_of((i - r * 8) * 16, 16)
        rows0_v[r, pl.ds(o, 16)] = jnp.zeros((16,), jnp.float32)
        return 0
    lax.fori_loop(0, CH * (D // 16), zrow, 0)

    def zcopy(i, _):
        o = pl.multiple_of(s * RPT + i * CH, CH)
        pltpu.sync_copy(rows0_v, acc_sh.at[pl.ds(o, CH)])
        return 0
    lax.fori_loop(0, RPT // CH, zcopy, 0)
    plsc.subcore_barrier()

    bufs = ((rows0_v, sem0), (rows1_v, sem1))

    def gather(j, rows, sem):
        o = pl.multiple_of(j * CH, CH)
        return pltpu.async_copy(hs_hbm.at[src_v.at[pl.ds(o, CH)]], rows, sem)

    gather(0, *bufs[0])
    gather(1, *bufs[1])

    def body(i, _):
        for b, (rows, sem) in enumerate(bufs):
            j = i * 2 + b
            pltpu.make_async_copy(hs_hbm.at[src_v.at[pl.ds(0, CH)]], rows, sem).wait()
            pltpu.sync_copy(rows, acc_sh.at[dst_v.at[j]], add=True)

            @pl.when(j + 2 < NCH)
            def _():
                gather(j + 2, rows, sem)
        return 0
    lax.fori_loop(0, NCH // 2, body, 0)

    plsc.subcore_barrier()
    a0 = pl.multiple_of(s * RPT, RPT)
    pltpu.sync_copy(acc_sh.at[pl.ds(a0, RPT)], out_hbm.at[c, pl.ds(a0, RPT)])


BN = 512
GRID = NP // BN


def _mm1_body(degp_ref, x_ref, w_ref, hs_ref, dis_ref):
    degp = degp_ref[...]
    ones = jnp.ones((NC, 1), jnp.float32)
    deg = lax.dot_general(degp, ones, (((0,), (0,)), ((), ())),
                          preferred_element_type=jnp.float32) + 1.0
    dis = lax.rsqrt(deg)
    h = jnp.dot(x_ref[...], w_ref[...], preferred_element_type=jnp.float32)
    hs_ref[...] = h * dis
    dis_ref[...] = dis


def _mm2_body(a0_ref, a1_ref, hs_ref, dis_ref, b_ref, w_ref, hs2_ref):
    dis = dis_ref[...]
    out1 = dis * (a0_ref[...] + a1_ref[...] + hs_ref[...]) + b_ref[...]
    h2 = jnp.dot(out1, w_ref[...], preferred_element_type=jnp.float32)
    hs2_ref[...] = h2 * dis


def _fin_body(a0_ref, a1_ref, hs_ref, dis_ref, b_ref, o_ref):
    o_ref[...] = (dis_ref[...] * (a0_ref[...] + a1_ref[...] + hs_ref[...])
                  + b_ref[...])


_row_spec = pl.BlockSpec((BN, D), lambda i: (i, 0))
_dis_spec = pl.BlockSpec((BN, 1), lambda i: (i, 0))
_w_spec = pl.BlockSpec((D, D), lambda i: (0, 0))
_b_spec = pl.BlockSpec((1, D), lambda i: (0, 0))

_mm1 = pl.pallas_call(
    _mm1_body,
    grid=(GRID,),
    in_specs=[pl.BlockSpec((NC, BN), lambda i: (0, i)), _row_spec, _w_spec],
    out_specs=[_row_spec, _dis_spec],
    out_shape=[jax.ShapeDtypeStruct((NP, D), jnp.float32),
               jax.ShapeDtypeStruct((NP, 1), jnp.float32)],
)

_mm2 = pl.pallas_call(
    _mm2_body,
    grid=(GRID,),
    in_specs=[_row_spec, _row_spec, _row_spec, _dis_spec, _b_spec, _w_spec],
    out_specs=_row_spec,
    out_shape=jax.ShapeDtypeStruct((NP, D), jnp.float32),
)

_BF = 400
_fin_row = pl.BlockSpec((_BF, D), lambda i: (i, 0))
_fin = pl.pallas_call(
    _fin_body,
    grid=(N // _BF,),
    in_specs=[_fin_row, _fin_row, _fin_row,
              pl.BlockSpec((_BF, 1), lambda i: (i, 0)), _b_spec],
    out_specs=_fin_row,
    out_shape=jax.ShapeDtypeStruct((N, D), jnp.float32),
)


def kernel(embeddings, edge_index, W1, b1, W2, b2):
    pad_idx = N + (jnp.arange(EP - E, dtype=jnp.int32) % (NP - N))
    src = jnp.concatenate([edge_index[0], pad_idx])
    dst2d = jnp.concatenate([edge_index[1], pad_idx]).reshape(EP // CH, CH)
    xpad = jnp.pad(embeddings, ((0, NP - N), (0, 0)))

    degp = _deg_call(dst2d)
    hs1, dis = _mm1(degp, xpad, W1)
    acc1 = _agg_call(hs1, src, dst2d)
    hs2 = _mm2(acc1[0], acc1[1], hs1, dis, b1.reshape(1, D), W2)
    acc2 = _agg_call(hs2, src, dst2d)
    return _fin(acc2[0], acc2[1], hs2, dis, b2.reshape(1, D))


# speedup vs baseline: 30.2844x; 1.0889x over previous
import functools

import jax
import jax.numpy as jnp
from jax import lax
from jax.experimental import pallas as pl
from jax.experimental.pallas import tpu as pltpu
from jax.experimental.pallas import tpu_sc as plsc

N = 10000
D = 128
E = 320000

NC = 2
NS = 16
NW = NC * NS
NP = 10240
CH = 128
EPT = 10240
EP = NW * EPT
NCH = EPT // CH
IB = 4
RPT = NP // NS

_MESH = plsc.VectorSubcoreMesh(core_axis_name="c", subcore_axis_name="s")


@functools.partial(
    pl.kernel,
    out_type=jax.ShapeDtypeStruct((NC, NP), jnp.float32),
    mesh=_MESH,
    scratch_types=[
        pltpu.VMEM((NCH, CH), jnp.int32),
        pltpu.VMEM((CH,), jnp.float32),
        pltpu.VMEM((RPT,), jnp.float32),
        pltpu.VMEM_SHARED((NP,), jnp.float32),
    ],
)
def _deg_call(dst_hbm, out_hbm, dst_v, ones_v, zeros_v, acc_sh):
    c = lax.axis_index("c")
    s = lax.axis_index("s")
    wid = c * NS + s

    row0 = pl.multiple_of(wid * NCH, NCH)
    pltpu.sync_copy(dst_hbm.at[pl.ds(row0, NCH)], dst_v)

    def fill(i, _):
        o = pl.multiple_of(i * 16, 16)
        ones_v[pl.ds(o, 16)] = jnp.ones((16,), jnp.float32)
        return 0
    lax.fori_loop(0, CH // 16, fill, 0)

    def zfill(i, _):
        o = pl.multiple_of(i * 16, 16)
        zeros_v[pl.ds(o, 16)] = jnp.zeros((16,), jnp.float32)
        return 0
    lax.fori_loop(0, RPT // 16, zfill, 0)

    a0 = pl.multiple_of(s * RPT, RPT)
    pltpu.sync_copy(zeros_v, acc_sh.at[pl.ds(a0, RPT)])
    plsc.subcore_barrier()

    def body(j, _):
        pltpu.sync_copy(ones_v, acc_sh.at[dst_v.at[j]], add=True)
        return 0
    lax.fori_loop(0, NCH, body, 0)

    plsc.subcore_barrier()
    pltpu.sync_copy(acc_sh.at[pl.ds(a0, RPT)], out_hbm.at[c, pl.ds(a0, RPT)])


@functools.partial(
    pl.kernel,
    out_type=jax.ShapeDtypeStruct((NC, NP, D), jnp.float32),
    mesh=_MESH,
    scratch_types=[
        pltpu.VMEM((IB, CH), jnp.int32),
        pltpu.VMEM((NCH, CH), jnp.int32),
        pltpu.VMEM((CH, D), jnp.float32),
        pltpu.VMEM((CH, D), jnp.float32),
        pltpu.VMEM_SHARED((NP, D), jnp.float32),
        [pltpu.SemaphoreType.DMA] * IB,
        pltpu.SemaphoreType.DMA,
        pltpu.SemaphoreType.DMA,
    ],
)
def _agg_call(hs_hbm, src_hbm, dst_hbm, out_hbm,
              srcr_v, dst_v, rows0_v, rows1_v, acc_sh, isems, sem0, sem1):
    c = lax.axis_index("c")
    s = lax.axis_index("s")
    wid = c * NS + s

    e0 = wid * EPT
    row0 = pl.multiple_of(wid * NCH, NCH)
    pltpu.sync_copy(dst_hbm.at[pl.ds(row0, NCH)], dst_v)

    def zrow(i, _):
        r = i // 8
        o = pl.multiple_of((i - r * 8) * 16, 16)
        rows0_v[r, pl.ds(o, 16)] = jnp.zeros((16,), jnp.float32)
        return 0
    lax.fori_loop(0, CH * (D // 16), zrow, 0)

    def zcopy(i, _):
        o = pl.multiple_of(s * RPT + i * CH, CH)
        pltpu.sync_copy(rows0_v, acc_sh.at[pl.ds(o, CH)])
        return 0
    lax.fori_loop(0, RPT // CH, zcopy, 0)
    plsc.subcore_barrier()

    rbufs = ((rows0_v, sem0), (rows1_v, sem1))

    def iload(j, slot):
        o = pl.multiple_of(e0 + j * CH, CH)
        pltpu.async_copy(src_hbm.at[pl.ds(o, CH)], srcr_v.at[slot], isems[slot])

    def iwait(slot):
        pltpu.make_async_copy(src_hbm.at[pl.ds(0, CH)], srcr_v.at[slot],
                              isems[slot]).wait()

    def gather(slot, rows, sem):
        pltpu.async_copy(hs_hbm.at[srcr_v.at[slot]], rows, sem)

    def gwait(rows, sem):
        pltpu.make_async_copy(hs_hbm.at[srcr_v.at[0]], rows, sem).wait()

    for k in range(IB):
        iload(k, k)
    iwait(0)
    gather(0, *rbufs[0])

    def body(i, _):
        for b in range(IB):
            j = i * IB + b
            rows, sem = rbufs[b % 2]
            nrows, nsem = rbufs[(b + 1) % 2]

            @pl.when(j + 1 < NCH)
            def _():
                iwait((b + 1) % IB)
                gather((b + 1) % IB, nrows, nsem)

            gwait(rows, sem)
            pltpu.sync_copy(rows, acc_sh.at[dst_v.at[j]], add=True)

            @pl.when(j + IB < NCH)
            def _():
                iload(j + IB, b)
        return 0
    lax.fori_loop(0, NCH // IB, body, 0)

    plsc.subcore_barrier()
    a0 = pl.multiple_of(s * RPT, RPT)
    pltpu.sync_copy(acc_sh.at[pl.ds(a0, RPT)], out_hbm.at[c, pl.ds(a0, RPT)])


BN = 512
GRID = NP // BN


def _mm1_body(degp_ref, x_ref, w_ref, hs_ref, dis_ref):
    degp = degp_ref[...]
    ones = jnp.ones((NC, 1), jnp.float32)
    deg = lax.dot_general(degp, ones, (((0,), (0,)), ((), ())),
                          preferred_element_type=jnp.float32) + 1.0
    dis = lax.rsqrt(deg)
    h = jnp.dot(x_ref[...], w_ref[...], preferred_element_type=jnp.float32)
    hs_ref[...] = h * dis
    dis_ref[...] = dis


def _mm2_body(a0_ref, a1_ref, hs_ref, dis_ref, b_ref, w_ref, hs2_ref):
    dis = dis_ref[...]
    out1 = dis * (a0_ref[...] + a1_ref[...] + hs_ref[...]) + b_ref[...]
    h2 = jnp.dot(out1, w_ref[...], preferred_element_type=jnp.float32)
    hs2_ref[...] = h2 * dis


def _fin_body(a0_ref, a1_ref, hs_ref, dis_ref, b_ref, o_ref):
    o_ref[...] = (dis_ref[...] * (a0_ref[...] + a1_ref[...] + hs_ref[...])
                  + b_ref[...])


_row_spec = pl.BlockSpec((BN, D), lambda i: (i, 0))
_dis_spec = pl.BlockSpec((BN, 1), lambda i: (i, 0))
_w_spec = pl.BlockSpec((D, D), lambda i: (0, 0))
_b_spec = pl.BlockSpec((1, D), lambda i: (0, 0))

_mm1 = pl.pallas_call(
    _mm1_body,
    grid=(GRID,),
    in_specs=[pl.BlockSpec((NC, BN), lambda i: (0, i)), _row_spec, _w_spec],
    out_specs=[_row_spec, _dis_spec],
    out_shape=[jax.ShapeDtypeStruct((NP, D), jnp.float32),
               jax.ShapeDtypeStruct((NP, 1), jnp.float32)],
)

_mm2 = pl.pallas_call(
    _mm2_body,
    grid=(GRID,),
    in_specs=[_row_spec, _row_spec, _row_spec, _dis_spec, _b_spec, _w_spec],
    out_specs=_row_spec,
    out_shape=jax.ShapeDtypeStruct((NP, D), jnp.float32),
)

_BF = 400
_fin_row = pl.BlockSpec((_BF, D), lambda i: (i, 0))
_fin = pl.pallas_call(
    _fin_body,
    grid=(N // _BF,),
    in_specs=[_fin_row, _fin_row, _fin_row,
              pl.BlockSpec((_BF, 1), lambda i: (i, 0)), _b_spec],
    out_specs=_fin_row,
    out_shape=jax.ShapeDtypeStruct((N, D), jnp.float32),
)


def kernel(embeddings, edge_index, W1, b1, W2, b2):
    pad_idx = N + (jnp.arange(EP - E, dtype=jnp.int32) % (NP - N))
    src = jnp.concatenate([edge_index[0], pad_idx])
    dst2d = jnp.concatenate([edge_index[1], pad_idx]).reshape(EP // CH, CH)
    xpad = jnp.pad(embeddings, ((0, NP - N), (0, 0)))

    degp = _deg_call(dst2d)
    hs1, dis = _mm1(degp, xpad, W1)
    acc1 = _agg_call(hs1, src, dst2d)
    hs2 = _mm2(acc1[0], acc1[1], hs1, dis, b1.reshape(1, D), W2)
    acc2 = _agg_call(hs2, src, dst2d)
    return _fin(acc2[0], acc2[1], hs2, dis, b2.reshape(1, D))


# speedup vs baseline: 31.6145x; 1.0439x over previous
import functools

import jax
import jax.numpy as jnp
from jax import lax
from jax.experimental import pallas as pl
from jax.experimental.pallas import tpu as pltpu
from jax.experimental.pallas import tpu_sc as plsc

N = 10000
D = 128
E = 320000

NC = 2
NS = 16
NW = NC * NS
NP = 10240
CH = 128
EPT = 10240
EP = NW * EPT
NCH = EPT // CH
IB = 4
RPT = NP // NS

_MESH = plsc.VectorSubcoreMesh(core_axis_name="c", subcore_axis_name="s")


@functools.partial(
    pl.kernel,
    out_type=jax.ShapeDtypeStruct((NC, NP), jnp.float32),
    mesh=_MESH,
    scratch_types=[
        pltpu.VMEM((NCH, CH), jnp.int32),
        pltpu.VMEM((CH,), jnp.float32),
        pltpu.VMEM((RPT,), jnp.float32),
        pltpu.VMEM_SHARED((NP,), jnp.float32),
    ],
)
def _deg_call(dst_hbm, out_hbm, dst_v, ones_v, zeros_v, acc_sh):
    c = lax.axis_index("c")
    s = lax.axis_index("s")
    wid = c * NS + s

    row0 = pl.multiple_of(wid * NCH, NCH)
    pltpu.sync_copy(dst_hbm.at[pl.ds(row0, NCH)], dst_v)

    def fill(i, _):
        o = pl.multiple_of(i * 16, 16)
        ones_v[pl.ds(o, 16)] = jnp.ones((16,), jnp.float32)
        return 0
    lax.fori_loop(0, CH // 16, fill, 0)

    def zfill(i, _):
        o = pl.multiple_of(i * 16, 16)
        zeros_v[pl.ds(o, 16)] = jnp.zeros((16,), jnp.float32)
        return 0
    lax.fori_loop(0, RPT // 16, zfill, 0)

    a0 = pl.multiple_of(s * RPT, RPT)
    pltpu.sync_copy(zeros_v, acc_sh.at[pl.ds(a0, RPT)])
    plsc.subcore_barrier()

    def body(j, _):
        pltpu.sync_copy(ones_v, acc_sh.at[dst_v.at[j]], add=True)
        return 0
    lax.fori_loop(0, NCH, body, 0)

    plsc.subcore_barrier()
    pltpu.sync_copy(acc_sh.at[pl.ds(a0, RPT)], out_hbm.at[c, pl.ds(a0, RPT)])


@functools.partial(
    pl.kernel,
    out_type=jax.ShapeDtypeStruct((NC, NP, D), jnp.float32),
    mesh=_MESH,
    scratch_types=[
        pltpu.VMEM((IB, CH), jnp.int32),
        pltpu.VMEM((NCH, CH), jnp.int32),
        pltpu.VMEM((CH, D), jnp.float32),
        pltpu.VMEM((CH, D), jnp.float32),
        pltpu.VMEM_SHARED((NP, D), jnp.float32),
        [pltpu.SemaphoreType.DMA] * IB,
        pltpu.SemaphoreType.DMA,
        pltpu.SemaphoreType.DMA,
    ],
)
def _agg_call(hs_hbm, src_hbm, dst_hbm, out_hbm,
              srcr_v, dst_v, rows0_v, rows1_v, acc_sh, isems, sem0, sem1):
    c = lax.axis_index("c")
    s = lax.axis_index("s")
    wid = c * NS + s

    e0 = wid * EPT
    row0 = pl.multiple_of(wid * NCH, NCH)
    pltpu.sync_copy(dst_hbm.at[pl.ds(row0, NCH)], dst_v)

    def zrow(i, _):
        r = i // 8
        o = pl.multiple_of((i - r * 8) * 16, 16)
        rows0_v[r, pl.ds(o, 16)] = jnp.zeros((16,), jnp.float32)
        return 0
    lax.fori_loop(0, CH * (D // 16), zrow, 0)

    def zcopy(i, _):
        o = pl.multiple_of(s * RPT + i * CH, CH)
        pltpu.sync_copy(rows0_v, acc_sh.at[pl.ds(o, CH)])
        return 0
    lax.fori_loop(0, RPT // CH, zcopy, 0)
    plsc.subcore_barrier()

    rbufs = ((rows0_v, sem0), (rows1_v, sem1))

    def iload(j, slot):
        o = pl.multiple_of(e0 + j * CH, CH)
        pltpu.async_copy(src_hbm.at[pl.ds(o, CH)], srcr_v.at[slot], isems[slot])

    def iwait(slot):
        pltpu.make_async_copy(src_hbm.at[pl.ds(0, CH)], srcr_v.at[slot],
                              isems[slot]).wait()

    def gather(slot, rows, sem):
        pltpu.async_copy(hs_hbm.at[srcr_v.at[slot]], rows, sem)

    def gwait(rows, sem):
        pltpu.make_async_copy(hs_hbm.at[srcr_v.at[0]], rows, sem).wait()

    for k in range(IB):
        iload(k, k)
    iwait(0)
    gather(0, *rbufs[0])

    def body(i, _):
        for b in range(IB):
            j = i * IB + b
            rows, sem = rbufs[b % 2]
            nrows, nsem = rbufs[(b + 1) % 2]

            @pl.when(j + 1 < NCH)
            def _():
                iwait((b + 1) % IB)
                gather((b + 1) % IB, nrows, nsem)

            gwait(rows, sem)
            pltpu.sync_copy(rows, acc_sh.at[dst_v.at[j]], add=True)

            @pl.when(j + IB < NCH)
            def _():
                iload(j + IB, b)
        return 0
    lax.fori_loop(0, NCH // IB, body, 0)

    plsc.subcore_barrier()
    a0 = pl.multiple_of(s * RPT, RPT)
    pltpu.sync_copy(acc_sh.at[pl.ds(a0, RPT)], out_hbm.at[c, pl.ds(a0, RPT)])


BN = 2048
GRID = NP // BN


def _mm1_body(degp_ref, x_ref, w_ref, hs_ref, dis_ref):
    degp = degp_ref[...]
    ones = jnp.ones((NC, 1), jnp.float32)
    deg = lax.dot_general(degp, ones, (((0,), (0,)), ((), ())),
                          preferred_element_type=jnp.float32) + 1.0
    dis = lax.rsqrt(deg)
    h = jnp.dot(x_ref[...], w_ref[...], preferred_element_type=jnp.float32)
    hs_ref[...] = h * dis
    dis_ref[...] = dis


def _mm2_body(a0_ref, a1_ref, hs_ref, dis_ref, b_ref, w_ref, hs2_ref):
    dis = dis_ref[...]
    out1 = dis * (a0_ref[...] + a1_ref[...] + hs_ref[...]) + b_ref[...]
    h2 = jnp.dot(out1, w_ref[...], preferred_element_type=jnp.float32)
    hs2_ref[...] = h2 * dis


def _fin_body(a0_ref, a1_ref, hs_ref, dis_ref, b_ref, o_ref):
    o_ref[...] = (dis_ref[...] * (a0_ref[...] + a1_ref[...] + hs_ref[...])
                  + b_ref[...])


_row_spec = pl.BlockSpec((BN, D), lambda i: (i, 0))
_dis_spec = pl.BlockSpec((BN, 1), lambda i: (i, 0))
_w_spec = pl.BlockSpec((D, D), lambda i: (0, 0))
_b_spec = pl.BlockSpec((1, D), lambda i: (0, 0))

_mm1 = pl.pallas_call(
    _mm1_body,
    grid=(GRID,),
    in_specs=[pl.BlockSpec((NC, BN), lambda i: (0, i)), _row_spec, _w_spec],
    out_specs=[_row_spec, _dis_spec],
    out_shape=[jax.ShapeDtypeStruct((NP, D), jnp.float32),
               jax.ShapeDtypeStruct((NP, 1), jnp.float32)],
)

_mm2 = pl.pallas_call(
    _mm2_body,
    grid=(GRID,),
    in_specs=[_row_spec, _row_spec, _row_spec, _dis_spec, _b_spec, _w_spec],
    out_specs=_row_spec,
    out_shape=jax.ShapeDtypeStruct((NP, D), jnp.float32),
)

_BF = 400
_fin_row = pl.BlockSpec((_BF, D), lambda i: (i, 0))
_fin = pl.pallas_call(
    _fin_body,
    grid=(N // _BF,),
    in_specs=[_fin_row, _fin_row, _fin_row,
              pl.BlockSpec((_BF, 1), lambda i: (i, 0)), _b_spec],
    out_specs=_fin_row,
    out_shape=jax.ShapeDtypeStruct((N, D), jnp.float32),
)


def kernel(embeddings, edge_index, W1, b1, W2, b2):
    pad_idx = N + (jnp.arange(EP - E, dtype=jnp.int32) % (NP - N))
    src = jnp.concatenate([edge_index[0], pad_idx])
    dst2d = jnp.concatenate([edge_index[1], pad_idx]).reshape(EP // CH, CH)
    xpad = jnp.pad(embeddings, ((0, NP - N), (0, 0)))

    degp = _deg_call(dst2d)
    hs1, dis = _mm1(degp, xpad, W1)
    acc1 = _agg_call(hs1, src, dst2d)
    hs2 = _mm2(acc1[0], acc1[1], hs1, dis, b1.reshape(1, D), W2)
    acc2 = _agg_call(hs2, src, dst2d)
    return _fin(acc2[0], acc2[1], hs2, dis, b2.reshape(1, D))


# speedup vs baseline: 32.6201x; 1.0318x over previous
import functools

import jax
import jax.numpy as jnp
from jax import lax
from jax.experimental import pallas as pl
from jax.experimental.pallas import tpu as pltpu
from jax.experimental.pallas import tpu_sc as plsc

N = 10000
D = 128
E = 320000

NC = 2
NS = 16
NW = NC * NS
NP = 10240
CH = 128
EPT = 10240
EP = NW * EPT
NCH = EPT // CH
IB = 8
RPT = NP // NS

_MESH = plsc.VectorSubcoreMesh(core_axis_name="c", subcore_axis_name="s")


@functools.partial(
    pl.kernel,
    out_type=jax.ShapeDtypeStruct((NC, NP), jnp.float32),
    mesh=_MESH,
    scratch_types=[
        pltpu.VMEM((NCH, CH), jnp.int32),
        pltpu.VMEM((CH,), jnp.float32),
        pltpu.VMEM((RPT,), jnp.float32),
        pltpu.VMEM_SHARED((NP,), jnp.float32),
    ],
)
def _deg_call(dst_hbm, out_hbm, dst_v, ones_v, zeros_v, acc_sh):
    c = lax.axis_index("c")
    s = lax.axis_index("s")
    wid = c * NS + s

    row0 = pl.multiple_of(wid * NCH, NCH)
    pltpu.sync_copy(dst_hbm.at[pl.ds(row0, NCH)], dst_v)

    def fill(i, _):
        o = pl.multiple_of(i * 16, 16)
        ones_v[pl.ds(o, 16)] = jnp.ones((16,), jnp.float32)
        return 0
    lax.fori_loop(0, CH // 16, fill, 0)

    def zfill(i, _):
        o = pl.multiple_of(i * 16, 16)
        zeros_v[pl.ds(o, 16)] = jnp.zeros((16,), jnp.float32)
        return 0
    lax.fori_loop(0, RPT // 16, zfill, 0)

    a0 = pl.multiple_of(s * RPT, RPT)
    pltpu.sync_copy(zeros_v, acc_sh.at[pl.ds(a0, RPT)])
    plsc.subcore_barrier()

    def body(j, _):
        pltpu.sync_copy(ones_v, acc_sh.at[dst_v.at[j]], add=True)
        return 0
    lax.fori_loop(0, NCH, body, 0)

    plsc.subcore_barrier()
    pltpu.sync_copy(acc_sh.at[pl.ds(a0, RPT)], out_hbm.at[c, pl.ds(a0, RPT)])


@functools.partial(
    pl.kernel,
    out_type=jax.ShapeDtypeStruct((NC, NP, D), jnp.float32),
    mesh=_MESH,
    scratch_types=[
        pltpu.VMEM((IB, CH), jnp.int32),
        pltpu.VMEM((NCH, CH), jnp.int32),
        pltpu.VMEM((CH, D), jnp.float32),
        pltpu.VMEM((CH, D), jnp.float32),
        pltpu.VMEM_SHARED((NP, D), jnp.float32),
        [pltpu.SemaphoreType.DMA] * IB,
        pltpu.SemaphoreType.DMA,
        pltpu.SemaphoreType.DMA,
    ],
)
def _agg_call(hs_hbm, src_hbm, dst_hbm, out_hbm,
              srcr_v, dst_v, rows0_v, rows1_v, acc_sh, isems, sem0, sem1):
    c = lax.axis_index("c")
    s = lax.axis_index("s")
    wid = c * NS + s

    e0 = wid * EPT
    row0 = pl.multiple_of(wid * NCH, NCH)
    pltpu.sync_copy(dst_hbm.at[pl.ds(row0, NCH)], dst_v)

    def zrow(i, _):
        r = i // 8
        o = pl.multiple_of((i - r * 8) * 16, 16)
        rows0_v[r, pl.ds(o, 16)] = jnp.zeros((16,), jnp.float32)
        return 0
    lax.fori_loop(0, CH * (D // 16), zrow, 0)

    def zcopy(i, _):
        o = pl.multiple_of(s * RPT + i * CH, CH)
        pltpu.sync_copy(rows0_v, acc_sh.at[pl.ds(o, CH)])
        return 0
    lax.fori_loop(0, RPT // CH, zcopy, 0)
    plsc.subcore_barrier()

    rbufs = ((rows0_v, sem0), (rows1_v, sem1))

    def iload(j, slot):
        o = pl.multiple_of(e0 + j * CH, CH)
        pltpu.async_copy(src_hbm.at[pl.ds(o, CH)], srcr_v.at[slot], isems[slot])

    def iwait(slot):
        pltpu.make_async_copy(src_hbm.at[pl.ds(0, CH)], srcr_v.at[slot],
                              isems[slot]).wait()

    def gather(slot, rows, sem):
        pltpu.async_copy(hs_hbm.at[srcr_v.at[slot]], rows, sem)

    def gwait(rows, sem):
        pltpu.make_async_copy(hs_hbm.at[srcr_v.at[0]], rows, sem).wait()

    for k in range(IB):
        iload(k, k)
    iwait(0)
    gather(0, *rbufs[0])

    def body(i, _):
        for b in range(IB):
            j = i * IB + b
            rows, sem = rbufs[b % 2]
            nrows, nsem = rbufs[(b + 1) % 2]

            @pl.when(j + 1 < NCH)
            def _():
                iwait((b + 1) % IB)
                gather((b + 1) % IB, nrows, nsem)

            gwait(rows, sem)
            pltpu.sync_copy(rows, acc_sh.at[dst_v.at[j]], add=True)

            @pl.when(j + IB < NCH)
            def _():
                iload(j + IB, b)
        return 0
    lax.fori_loop(0, NCH // IB, body, 0)

    plsc.subcore_barrier()
    a0 = pl.multiple_of(s * RPT, RPT)
    pltpu.sync_copy(acc_sh.at[pl.ds(a0, RPT)], out_hbm.at[c, pl.ds(a0, RPT)])


BN = 2048
GRID = NP // BN


def _mm1_body(degp_ref, x_ref, w_ref, hs_ref, dis_ref):
    degp = degp_ref[...]
    ones = jnp.ones((NC, 1), jnp.float32)
    deg = lax.dot_general(degp, ones, (((0,), (0,)), ((), ())),
                          preferred_element_type=jnp.float32) + 1.0
    dis = lax.rsqrt(deg)
    h = jnp.dot(x_ref[...], w_ref[...], preferred_element_type=jnp.float32)
    hs_ref[...] = h * dis
    dis_ref[...] = dis


def _mm2_body(a0_ref, a1_ref, hs_ref, dis_ref, b_ref, w_ref, hs2_ref):
    dis = dis_ref[...]
    out1 = dis * (a0_ref[...] + a1_ref[...] + hs_ref[...]) + b_ref[...]
    h2 = jnp.dot(out1, w_ref[...], preferred_element_type=jnp.float32)
    hs2_ref[...] = h2 * dis


def _fin_body(a0_ref, a1_ref, hs_ref, dis_ref, b_ref, o_ref):
    o_ref[...] = (dis_ref[...] * (a0_ref[...] + a1_ref[...] + hs_ref[...])
                  + b_ref[...])


_row_spec = pl.BlockSpec((BN, D), lambda i: (i, 0))
_dis_spec = pl.BlockSpec((BN, 1), lambda i: (i, 0))
_w_spec = pl.BlockSpec((D, D), lambda i: (0, 0))
_b_spec = pl.BlockSpec((1, D), lambda i: (0, 0))

_mm1 = pl.pallas_call(
    _mm1_body,
    grid=(GRID,),
    in_specs=[pl.BlockSpec((NC, BN), lambda i: (0, i)), _row_spec, _w_spec],
    out_specs=[_row_spec, _dis_spec],
    out_shape=[jax.ShapeDtypeStruct((NP, D), jnp.float32),
               jax.ShapeDtypeStruct((NP, 1), jnp.float32)],
)

_mm2 = pl.pallas_call(
    _mm2_body,
    grid=(GRID,),
    in_specs=[_row_spec, _row_spec, _row_spec, _dis_spec, _b_spec, _w_spec],
    out_specs=_row_spec,
    out_shape=jax.ShapeDtypeStruct((NP, D), jnp.float32),
)

_BF = 2000
_fin_row = pl.BlockSpec((_BF, D), lambda i: (i, 0))
_fin = pl.pallas_call(
    _fin_body,
    grid=(N // _BF,),
    in_specs=[_fin_row, _fin_row, _fin_row,
              pl.BlockSpec((_BF, 1), lambda i: (i, 0)), _b_spec],
    out_specs=_fin_row,
    out_shape=jax.ShapeDtypeStruct((N, D), jnp.float32),
)


def kernel(embeddings, edge_index, W1, b1, W2, b2):
    pad_idx = N + (jnp.arange(EP - E, dtype=jnp.int32) % (NP - N))
    src = jnp.concatenate([edge_index[0], pad_idx])
    dst2d = jnp.concatenate([edge_index[1], pad_idx]).reshape(EP // CH, CH)
    xpad = jnp.pad(embeddings, ((0, NP - N), (0, 0)))

    degp = _deg_call(dst2d)
    hs1, dis = _mm1(degp, xpad, W1)
    acc1 = _agg_call(hs1, src, dst2d)
    hs2 = _mm2(acc1[0], acc1[1], hs1, dis, b1.reshape(1, D), W2)
    acc2 = _agg_call(hs2, src, dst2d)
    return _fin(acc2[0], acc2[1], hs2, dis, b2.reshape(1, D))


# speedup vs baseline: 33.5162x; 1.0275x over previous
import functools

import jax
import jax.numpy as jnp
from jax import lax
from jax.experimental import pallas as pl
from jax.experimental.pallas import tpu as pltpu
from jax.experimental.pallas import tpu_sc as plsc

N = 10000
D = 128
E = 320000

NC = 2
NS = 16
NW = NC * NS
NP = 10240
CH = 128
EPT = 10240
EP = NW * EPT
NCH = EPT // CH
IB = 8
RPT = NP // NS

_MESH = plsc.VectorSubcoreMesh(core_axis_name="c", subcore_axis_name="s")


@functools.partial(
    pl.kernel,
    out_type=jax.ShapeDtypeStruct((NC, NP), jnp.float32),
    mesh=_MESH,
    scratch_types=[
        pltpu.VMEM((NCH, CH), jnp.int32),
        pltpu.VMEM((CH,), jnp.float32),
        pltpu.VMEM((RPT,), jnp.float32),
        pltpu.VMEM_SHARED((NP,), jnp.float32),
    ],
)
def _deg_call(dst_hbm, out_hbm, dst_v, ones_v, zeros_v, acc_sh):
    c = lax.axis_index("c")
    s = lax.axis_index("s")
    wid = c * NS + s

    row0 = pl.multiple_of(wid * NCH, NCH)
    pltpu.sync_copy(dst_hbm.at[pl.ds(row0, NCH)], dst_v)

    def fill(i, _):
        o = pl.multiple_of(i * 16, 16)
        ones_v[pl.ds(o, 16)] = jnp.ones((16,), jnp.float32)
        return 0
    lax.fori_loop(0, CH // 16, fill, 0)

    def zfill(i, _):
        o = pl.multiple_of(i * 16, 16)
        zeros_v[pl.ds(o, 16)] = jnp.zeros((16,), jnp.float32)
        return 0
    lax.fori_loop(0, RPT // 16, zfill, 0)

    a0 = pl.multiple_of(s * RPT, RPT)
    pltpu.sync_copy(zeros_v, acc_sh.at[pl.ds(a0, RPT)])
    plsc.subcore_barrier()

    def body(j, _):
        pltpu.sync_copy(ones_v, acc_sh.at[dst_v.at[j]], add=True)
        return 0
    lax.fori_loop(0, NCH, body, 0)

    plsc.subcore_barrier()
    pltpu.sync_copy(acc_sh.at[pl.ds(a0, RPT)], out_hbm.at[c, pl.ds(a0, RPT)])


CH2 = 64
NCH2 = EPT // CH2
NPR = NCH2 // 2
AIB = 8
AIB2 = AIB // 2


@functools.partial(
    pl.kernel,
    out_type=jax.ShapeDtypeStruct((NC, NP, D), jnp.float32),
    mesh=_MESH,
    scratch_types=[
        pltpu.VMEM((AIB2, CH), jnp.int32),
        pltpu.VMEM((NPR, CH), jnp.int32),
        pltpu.VMEM((CH2, D), jnp.float32),
        pltpu.VMEM((CH2, D), jnp.float32),
        pltpu.VMEM((CH2, D), jnp.float32),
        pltpu.VMEM((CH2, D), jnp.float32),
        pltpu.VMEM_SHARED((NP, D), jnp.float32),
        [pltpu.SemaphoreType.DMA] * AIB2,
        [pltpu.SemaphoreType.DMA] * 4,
        [pltpu.SemaphoreType.DMA] * 4,
    ],
)
def _agg_call(hs_hbm, src_hbm, dst_hbm, out_hbm,
              srcr_v, dst_v, rows0_v, rows1_v, rows2_v, rows3_v, acc_sh,
              isems, gsems, ssems):
    c = lax.axis_index("c")
    s = lax.axis_index("s")
    wid = c * NS + s

    e0 = wid * EPT
    row0 = pl.multiple_of(wid * NPR, NPR)
    pltpu.sync_copy(dst_hbm.at[pl.ds(row0, NPR)], dst_v)

    def zrow(i, _):
        r = i // 8
        o = pl.multiple_of((i - r * 8) * 16, 16)
        rows0_v[r, pl.ds(o, 16)] = jnp.zeros((16,), jnp.float32)
        return 0
    lax.fori_loop(0, CH2 * (D // 16), zrow, 0)

    def zcopy(i, _):
        o = pl.multiple_of(s * RPT + i * CH2, CH2)
        pltpu.sync_copy(rows0_v, acc_sh.at[pl.ds(o, CH2)])
        return 0
    lax.fori_loop(0, RPT // CH2, zcopy, 0)
    plsc.subcore_barrier()

    rbufs = (rows0_v, rows1_v, rows2_v, rows3_v)

    def iload(p, slot):
        o = pl.multiple_of(e0 + p * CH, CH)
        pltpu.async_copy(src_hbm.at[pl.ds(o, CH)], srcr_v.at[slot],
                         isems[slot])

    def iwait(slot):
        pltpu.make_async_copy(src_hbm.at[pl.ds(0, CH)], srcr_v.at[slot],
                              isems[slot]).wait()

    def gather(slot, half, buf):
        idx = srcr_v.at[slot, pl.ds(64 * half, CH2)]
        pltpu.async_copy(hs_hbm.at[idx], rbufs[buf], gsems[buf])

    def gwait(buf):
        pltpu.make_async_copy(hs_hbm.at[srcr_v.at[0, pl.ds(0, CH2)]],
                              rbufs[buf], gsems[buf]).wait()

    def swait(buf):
        pltpu.make_async_copy(rbufs[buf], acc_sh.at[pl.ds(0, CH2)],
                              ssems[buf]).wait()

    for p in range(AIB2):
        iload(p, p)
    iwait(0)
    gather(0, 0, 0)
    gather(0, 1, 1)

    def body(i, _):
        for b in range(AIB):
            j = i * AIB + b

            @pl.when((j >= 2) & (j + 2 < NCH2))
            def _():
                swait((b + 2) % 4)

            @pl.when(j + 2 < NCH2)
            def _():
                if b % 2 == 0:
                    iwait(((b + 2) // 2) % AIB2)
                gather(((b + 2) // 2) % AIB2, (b + 2) % 2, (b + 2) % 4)

            gwait(b % 4)
            dst = dst_v.at[lax.div(j, 2), pl.ds(64 * (b % 2), CH2)]
            pltpu.async_copy(rbufs[b % 4], acc_sh.at[dst],
                             ssems[b % 4], add=True)

            @pl.when((b % 2 == 1) & (j // 2 + AIB2 < NPR))
            def _():
                iload(j // 2 + AIB2, (b // 2) % AIB2)
        return 0
    lax.fori_loop(0, NCH2 // AIB, body, 0)

    for b in range(4):
        swait(b)

    plsc.subcore_barrier()
    a0 = pl.multiple_of(s * RPT, RPT)
    pltpu.sync_copy(acc_sh.at[pl.ds(a0, RPT)], out_hbm.at[c, pl.ds(a0, RPT)])


BN = 2048
GRID = NP // BN


def _mm1_body(degp_ref, x_ref, w_ref, hs_ref, dis_ref):
    degp = degp_ref[...]
    ones = jnp.ones((NC, 1), jnp.float32)
    deg = lax.dot_general(degp, ones, (((0,), (0,)), ((), ())),
                          preferred_element_type=jnp.float32) + 1.0
    dis = lax.rsqrt(deg)
    h = jnp.dot(x_ref[...], w_ref[...], preferred_element_type=jnp.float32)
    hs_ref[...] = h * dis
    dis_ref[...] = dis


def _mm2_body(a0_ref, a1_ref, hs_ref, dis_ref, b_ref, w_ref, hs2_ref):
    dis = dis_ref[...]
    out1 = dis * (a0_ref[...] + a1_ref[...] + hs_ref[...]) + b_ref[...]
    h2 = jnp.dot(out1, w_ref[...], preferred_element_type=jnp.float32)
    hs2_ref[...] = h2 * dis


def _fin_body(a0_ref, a1_ref, hs_ref, dis_ref, b_ref, o_ref):
    o_ref[...] = (dis_ref[...] * (a0_ref[...] + a1_ref[...] + hs_ref[...])
                  + b_ref[...])


_row_spec = pl.BlockSpec((BN, D), lambda i: (i, 0))
_dis_spec = pl.BlockSpec((BN, 1), lambda i: (i, 0))
_w_spec = pl.BlockSpec((D, D), lambda i: (0, 0))
_b_spec = pl.BlockSpec((1, D), lambda i: (0, 0))

_mm1 = pl.pallas_call(
    _mm1_body,
    grid=(GRID,),
    in_specs=[pl.BlockSpec((NC, BN), lambda i: (0, i)), _row_spec, _w_spec],
    out_specs=[_row_spec, _dis_spec],
    out_shape=[jax.ShapeDtypeStruct((NP, D), jnp.float32),
               jax.ShapeDtypeStruct((NP, 1), jnp.float32)],
)

_mm2 = pl.pallas_call(
    _mm2_body,
    grid=(GRID,),
    in_specs=[_row_spec, _row_spec, _row_spec, _dis_spec, _b_spec, _w_spec],
    out_specs=_row_spec,
    out_shape=jax.ShapeDtypeStruct((NP, D), jnp.float32),
)

_BF = 2000
_fin_row = pl.BlockSpec((_BF, D), lambda i: (i, 0))
_fin = pl.pallas_call(
    _fin_body,
    grid=(N // _BF,),
    in_specs=[_fin_row, _fin_row, _fin_row,
              pl.BlockSpec((_BF, 1), lambda i: (i, 0)), _b_spec],
    out_specs=_fin_row,
    out_shape=jax.ShapeDtypeStruct((N, D), jnp.float32),
)


def kernel(embeddings, edge_index, W1, b1, W2, b2):
    pad_idx = N + (jnp.arange(EP - E, dtype=jnp.int32) % (NP - N))
    src = jnp.concatenate([edge_index[0], pad_idx])
    dst2d = jnp.concatenate([edge_index[1], pad_idx]).reshape(EP // CH, CH)
    xpad = jnp.pad(embeddings, ((0, NP - N), (0, 0)))

    degp = _deg_call(dst2d)
    hs1, dis = _mm1(degp, xpad, W1)
    acc1 = _agg_call(hs1, src, dst2d)
    hs2 = _mm2(acc1[0], acc1[1], hs1, dis, b1.reshape(1, D), W2)
    acc2 = _agg_call(hs2, src, dst2d)
    return _fin(acc2[0], acc2[1], hs2, dis, b2.reshape(1, D))
